# baseline (speedup 1.0000x reference)
import functools

import jax
import jax.numpy as jnp
from jax import lax
from jax.experimental import pallas as pl
from jax.experimental.pallas import tpu as pltpu
from jax.experimental.pallas import tpu_sc as plsc

N = 1048576
NC = 2
NS = 16
NW = NC * NS
ROWS_PER_TILE = N // NW
R = 2048
CHUNKS = ROWS_PER_TILE // R
G = R // 16

_C3 = 1.0 / 3.0
_C5 = 1.0 / 5.0
_C7 = 1.0 / 7.0
_C9 = 1.0 / 9.0
_C11 = 1.0 / 11.0


def _group_body(pbuf, ybuf, bbuf):
    def body(g, accs):
        acc_bce, acc_m0, acc_box, acc_m1 = accs
        b16 = g * 16
        rows = jax.lax.iota(jnp.int32, 16) + b16
        prow = rows * 15
        brow = rows * 4
        y = ybuf[pl.ds(b16, 16)]
        x = plsc.load_gather(pbuf, [prow])
        ax = jnp.abs(x)
        u = jnp.exp(-ax)
        z = u / (u + 2.0)
        z2 = z * z
        poly = 1.0 + z2 * (_C3 + z2 * (_C5 + z2 * (_C7 + z2 * (_C9 + z2 * _C11))))
        sp = jnp.maximum(x, 0.0) + (2.0 * z) * poly
        m0 = jnp.where(y >= 0.0, 1.0, 0.0)
        acc_bce = acc_bce + (sp - y * x) * m0
        acc_m0 = acc_m0 + m0
        m1 = jnp.where(y != 0.0, 1.0, 0.0)
        s = None
        for c in range(4):
            pc = plsc.load_gather(pbuf, [prow + (c + 1)])
            bc = plsc.load_gather(bbuf, [brow + c])
            d = pc - bc
            s = d * d if s is None else s + d * d
        acc_box = acc_box + m1 * s
        acc_m1 = acc_m1 + m1
        return (acc_bce, acc_m0, acc_box, acc_m1)

    return body


@functools.partial(
    pl.kernel,
    out_type=jax.ShapeDtypeStruct((NW, 4 * 16), jnp.float32),
    mesh=plsc.VectorSubcoreMesh(core_axis_name="c", subcore_axis_name="s"),
    compiler_params=pltpu.CompilerParams(needs_layout_passes=False),
    scratch_types=[
        pltpu.VMEM((R * 15,), jnp.float32),
        pltpu.VMEM((R * 15,), jnp.float32),
        pltpu.VMEM((R,), jnp.float32),
        pltpu.VMEM((R,), jnp.float32),
        pltpu.VMEM((R * 4,), jnp.float32),
        pltpu.VMEM((R * 4,), jnp.float32),
        pltpu.VMEM((4 * 16,), jnp.float32),
        pltpu.SemaphoreType.DMA,
        pltpu.SemaphoreType.DMA,
    ],
)
def _loss_partials(pred_hbm, label_hbm, bbox_hbm, out_hbm,
                   pbuf0, pbuf1, ybuf0, ybuf1, bbuf0, bbuf1, obuf,
                   sem0, sem1):
    wid = lax.axis_index("s") * NC + lax.axis_index("c")
    tile_base = wid * ROWS_PER_TILE
    bufs = ((pbuf0, ybuf0, bbuf0, sem0), (pbuf1, ybuf1, bbuf1, sem1))

    def start(j, pbuf, ybuf, bbuf, sem):
        base = tile_base + j * R
        return (
            pltpu.async_copy(pred_hbm.at[pl.ds(base * 15, R * 15)], pbuf, sem),
            pltpu.async_copy(label_hbm.at[pl.ds(base, R)], ybuf, sem),
            pltpu.async_copy(bbox_hbm.at[pl.ds(base * 4, R * 4)], bbuf, sem),
        )

    zeros = jnp.zeros((16,), jnp.float32)
    accs = (zeros, zeros, zeros, zeros)
    handles = start(0, *bufs[0])
    for j in range(CHUNKS):
        pbuf, ybuf, bbuf, _ = bufs[j % 2]
        cur_handles = handles
        if j + 1 < CHUNKS:
            handles = start(j + 1, *bufs[(j + 1) % 2])
        for h in cur_handles:
            h.wait()
        accs = lax.fori_loop(0, G, _group_body(pbuf, ybuf, bbuf), accs)

    for i in range(4):
        obuf[pl.ds(i * 16, 16)] = accs[i]
    pltpu.sync_copy(obuf, out_hbm.at[wid])


def kernel(pred, gt_label, gt_bbox, gt_landmark):
    pred = pred.reshape(pred.shape[0], 15)
    parts = _loss_partials(pred.reshape(-1), gt_label, gt_bbox.reshape(-1))
    s = jnp.sum(parts.reshape(NW, 4, 16), axis=(0, 2))
    cls_loss = (s[0] / jnp.maximum(s[1], 1.0)) * 1.0
    box_loss = (s[2] / jnp.maximum(s[3] * 4.0, 1.0)) * 1.0
    landmark_loss = jnp.float32(0.0)
    total_loss = cls_loss + box_loss + landmark_loss
    return (total_loss, cls_loss, box_loss, landmark_loss)


# speedup vs baseline: 36.1369x; 36.1369x over previous
import functools

import jax
import jax.numpy as jnp
from jax import lax
from jax.experimental import pallas as pl
from jax.experimental.pallas import tpu as pltpu
from jax.experimental.pallas import tpu_sc as plsc

N = 1048576
NC = 2
NS = 16
NW = NC * NS
ROWS_PER_TILE = N // NW
R = 4096
CHUNKS = ROWS_PER_TILE // R
G = R // 16

_C3 = 1.0 / 3.0
_C5 = 1.0 / 5.0
_C7 = 1.0 / 7.0
_C9 = 1.0 / 9.0
_C11 = 1.0 / 11.0


def _group_body(bufs):
    xbuf, ybuf, p1, p2, p3, p4, b1, b2, b3, b4 = bufs

    def body(g, accs):
        acc_bce, acc_m0, acc_box, acc_m1 = accs
        ds = pl.ds(g * 16, 16)
        x = xbuf[ds]
        y = ybuf[ds]
        ax = jnp.abs(x)
        u = jnp.exp(-ax)
        z = u / (u + 2.0)
        z2 = z * z
        poly = 1.0 + z2 * (_C3 + z2 * (_C5 + z2 * (_C7 + z2 * (_C9 + z2 * _C11))))
        sp = jnp.maximum(x, 0.0) + (2.0 * z) * poly
        m0 = jnp.where(y >= 0.0, 1.0, 0.0)
        acc_bce = acc_bce + (sp - y * x) * m0
        acc_m0 = acc_m0 + m0
        m1 = jnp.where(y != 0.0, 1.0, 0.0)
        d1 = p1[ds] - b1[ds]
        d2 = p2[ds] - b2[ds]
        d3 = p3[ds] - b3[ds]
        d4 = p4[ds] - b4[ds]
        s = d1 * d1 + d2 * d2 + d3 * d3 + d4 * d4
        acc_box = acc_box + m1 * s
        acc_m1 = acc_m1 + m1
        return (acc_bce, acc_m0, acc_box, acc_m1)

    return body


@functools.partial(
    pl.kernel,
    out_type=jax.ShapeDtypeStruct((NW, 4 * 16), jnp.float32),
    mesh=plsc.VectorSubcoreMesh(core_axis_name="c", subcore_axis_name="s"),
    compiler_params=pltpu.CompilerParams(needs_layout_passes=False),
    scratch_types=(
        [pltpu.VMEM((R,), jnp.float32) for _ in range(20)]
        + [pltpu.VMEM((4 * 16,), jnp.float32)]
        + [pltpu.SemaphoreType.DMA, pltpu.SemaphoreType.DMA]
    ),
)
def _loss_partials(predt_hbm, label_hbm, bboxt_hbm, out_hbm, *scratch):
    bufsets = (scratch[0:10], scratch[10:20])
    obuf = scratch[20]
    sems = scratch[21:23]
    wid = lax.axis_index("s") * NC + lax.axis_index("c")
    tile_base = wid * ROWS_PER_TILE

    def start(j, bufs, sem):
        base = tile_base + j * R
        ds = pl.ds(base, R)
        handles = [
            pltpu.async_copy(predt_hbm.at[0, ds], bufs[0], sem),
            pltpu.async_copy(label_hbm.at[ds], bufs[1], sem),
        ]
        for c in range(4):
            handles.append(pltpu.async_copy(predt_hbm.at[c + 1, ds], bufs[2 + c], sem))
            handles.append(pltpu.async_copy(bboxt_hbm.at[c, ds], bufs[6 + c], sem))
        return handles

    zeros = jnp.zeros((16,), jnp.float32)
    accs = (zeros, zeros, zeros, zeros)
    handles = start(0, bufsets[0], sems[0])
    for j in range(CHUNKS):
        bufs = bufsets[j % 2]
        cur_handles = handles
        if j + 1 < CHUNKS:
            handles = start(j + 1, bufsets[(j + 1) % 2], sems[(j + 1) % 2])
        for h in cur_handles:
            h.wait()
        accs = lax.fori_loop(0, G, _group_body(bufs), accs)

    for i in range(4):
        obuf[pl.ds(i * 16, 16)] = accs[i]
    pltpu.sync_copy(obuf, out_hbm.at[wid])


def kernel(pred, gt_label, gt_bbox, gt_landmark):
    pred = pred.reshape(pred.shape[0], 15)
    parts = _loss_partials(pred.T, gt_label, gt_bbox.T)
    s = jnp.sum(parts.reshape(NW, 4, 16), axis=(0, 2))
    cls_loss = (s[0] / jnp.maximum(s[1], 1.0)) * 1.0
    box_loss = (s[2] / jnp.maximum(s[3] * 4.0, 1.0)) * 1.0
    landmark_loss = jnp.float32(0.0)
    total_loss = cls_loss + box_loss + landmark_loss
    return (total_loss, cls_loss, box_loss, landmark_loss)
